# Initial kernel scaffold; baseline (speedup 1.0000x reference)
import jax
import jax.numpy as jnp
from jax.experimental import pallas as pl


def kernel(x, edge_index, edge_attr, u, batch, W1a, b1a, g1, beta1, W1b, b1b, W2a, b2a, g2, beta2, W2b, b2b):
    raise NotImplementedError("write your pallas kernel here")


# speedup vs baseline: 2.1480x; 2.1480x over previous
import functools

import jax
import jax.numpy as jnp
from jax import lax
from jax.experimental import pallas as pl
from jax.experimental.pallas import tpu as pltpu
from jax.experimental.pallas import tpu_sc as plsc

N = 10000
E = 320000
D = 128
DE = 16
H = 128

NC = 2
NS = 16
NW = NC * NS

LANE = 128
SUB = 4
BLK = SUB * LANE
NB = E // BLK
NHALF = N // NC
ZR = 1000
TRASH = NHALF

TE = 2000
NEB = E // TE

_sc_mesh = plsc.VectorSubcoreMesh(core_axis_name="c", subcore_axis_name="s")


@functools.partial(
    pl.kernel,
    mesh=_sc_mesh,
    out_type=jax.ShapeDtypeStruct((E, H), jnp.float32),
    scratch_types=[
        pltpu.VMEM((BLK,), jnp.int32),
        pltpu.VMEM((BLK, H), jnp.float32),
        pltpu.SemaphoreType.DMA,
    ],
)
def _sc_gather(p_hbm, row_hbm, out_hbm, idx_v, rows_v, sem):
    wid = lax.axis_index("s") * NC + lax.axis_index("c")
    nb = (NB - wid + NW - 1) // NW

    def body(i, _):
        base = (wid + i * NW) * BLK
        pltpu.sync_copy(row_hbm.at[pl.ds(base, BLK)], idx_v)
        for j in range(SUB):
            pltpu.async_copy(
                p_hbm.at[idx_v.at[pl.ds(j * LANE, LANE)]],
                rows_v.at[pl.ds(j * LANE, LANE)],
                sem,
            ).wait()
        pltpu.sync_copy(rows_v, out_hbm.at[pl.ds(base, BLK)])
        return 0

    lax.fori_loop(0, nb, body, 0)


@functools.partial(
    pl.kernel,
    mesh=_sc_mesh,
    out_type=jax.ShapeDtypeStruct((N, H), jnp.float32),
    scratch_types=[
        pltpu.VMEM((BLK,), jnp.int32),
        pltpu.VMEM((LANE,), jnp.int32),
        pltpu.VMEM((BLK, H), jnp.float32),
        pltpu.VMEM_SHARED((NHALF + 8, H), jnp.float32),
    ],
)
def _sc_scatter(zeros_hbm, col_hbm, dat_hbm, out_hbm, idx_v, idx2_v, dat_v,
                acc_sh):
    cid = lax.axis_index("c")
    sid = lax.axis_index("s")
    lo = cid * NHALF

    r0 = sid * ZR

    @pl.when(sid < NHALF // ZR)
    def _():
        pltpu.sync_copy(zeros_hbm, acc_sh.at[pl.ds(r0, ZR)])

    @pl.when(sid == NHALF // ZR)
    def _():
        pltpu.sync_copy(zeros_hbm.at[pl.ds(0, 8)],
                        acc_sh.at[pl.ds(NHALF, 8)])

    plsc.subcore_barrier()

    nb = (NB - sid + NS - 1) // NS

    def body(i, _):
        base = (sid + i * NS) * BLK
        pltpu.sync_copy(col_hbm.at[pl.ds(base, BLK)], idx_v)
        pltpu.sync_copy(dat_hbm.at[pl.ds(base, BLK)], dat_v)
        for j in range(SUB):
            for k in range(LANE // 16):
                v = idx_v[pl.ds(j * LANE + k * 16, 16)]
                in_range = (v >= lo) & (v < lo + NHALF)
                idx2_v[pl.ds(k * 16, 16)] = jnp.where(in_range, v - lo, TRASH)
            pltpu.sync_copy(dat_v.at[pl.ds(j * LANE, LANE)],
                            acc_sh.at[idx2_v], add=True)
        return 0

    lax.fori_loop(0, nb, body, 0)
    plsc.subcore_barrier()

    @pl.when(sid < NHALF // ZR)
    def _():
        pltpu.sync_copy(acc_sh.at[pl.ds(r0, ZR)],
                        out_hbm.at[pl.ds(lo + r0, ZR)])


def _prep_body(x_ref, w_ref, b_ref, p_ref):
    p_ref[...] = (
        jnp.dot(x_ref[...], w_ref[...], preferred_element_type=jnp.float32)
        + b_ref[...]
    )


def _stats_body(g_ref, ea_ref, w_ref, g1_ref, beta1_ref, a_ref, c_ref, acc):
    i = pl.program_id(0)
    h = g_ref[...] + jnp.dot(ea_ref[...], w_ref[...],
                             preferred_element_type=jnp.float32)
    s1 = jnp.sum(h, axis=0, keepdims=True)
    s2 = jnp.sum(h * h, axis=0, keepdims=True)

    @pl.when(i == 0)
    def _():
        acc[0:1, :] = s1
        acc[1:2, :] = s2

    @pl.when(i > 0)
    def _():
        acc[0:1, :] += s1
        acc[1:2, :] += s2

    @pl.when(i == NEB - 1)
    def _():
        mu = acc[0:1, :] * (1.0 / E)
        var = acc[1:2, :] * (1.0 / E) - mu * mu
        a = g1_ref[...] * lax.rsqrt(var + 1e-5)
        a_ref[...] = a
        c_ref[...] = beta1_ref[...] - mu * a


def _edge_body(g_ref, ea_ref, w_ref, a_ref, c_ref, w1b_ref, b1b_ref, o_ref):
    h = g_ref[...] + jnp.dot(ea_ref[...], w_ref[...],
                             preferred_element_type=jnp.float32)
    h = jnp.maximum(h * a_ref[...] + c_ref[...], 0.0)
    o_ref[...] = (
        jnp.dot(h, w1b_ref[...], preferred_element_type=jnp.float32)
        + b1b_ref[...]
    )


def _node_body(x_ref, agg_ref, w2x_ref, w2a_ref, b2a_ref, g2_ref,
               beta2_ref, w2b_ref, b2b_ref, o_ref):
    agg = agg_ref[...]
    t = (
        jnp.dot(x_ref[...], w2x_ref[...], preferred_element_type=jnp.float32)
        + jnp.dot(agg, w2a_ref[...], preferred_element_type=jnp.float32)
        + b2a_ref[...]
    )
    mu = jnp.mean(t, axis=0, keepdims=True)
    var = jnp.mean(t * t, axis=0, keepdims=True) - mu * mu
    h = jnp.maximum((t - mu) * lax.rsqrt(var + 1e-5) * g2_ref[...]
                    + beta2_ref[...], 0.0)
    o_ref[...] = (
        jnp.dot(h, w2b_ref[...], preferred_element_type=jnp.float32)
        + b2b_ref[...]
    )


def kernel(x, edge_index, edge_attr, u, batch,
           W1a, b1a, g1, beta1, W1b, b1b,
           W2a, b2a, g2, beta2, W2b, b2b):
    row = edge_index[0]
    col = edge_index[1]

    w1x = W1a[:D]
    w1e = W1a[D:]
    b1a2 = b1a.reshape(1, H)
    g1_2 = g1.reshape(1, H)
    beta1_2 = beta1.reshape(1, H)

    p = pl.pallas_call(
        _prep_body,
        out_shape=jax.ShapeDtypeStruct((N, H), jnp.float32),
    )(x, w1x, b1a2)

    g = _sc_gather(p, row)

    a, c = pl.pallas_call(
        _stats_body,
        grid=(NEB,),
        in_specs=[
            pl.BlockSpec((TE, H), lambda i: (i, 0)),
            pl.BlockSpec((TE, DE), lambda i: (i, 0)),
            pl.BlockSpec((DE, H), lambda i: (0, 0)),
            pl.BlockSpec((1, H), lambda i: (0, 0)),
            pl.BlockSpec((1, H), lambda i: (0, 0)),
        ],
        out_specs=[
            pl.BlockSpec((1, H), lambda i: (0, 0)),
            pl.BlockSpec((1, H), lambda i: (0, 0)),
        ],
        out_shape=[
            jax.ShapeDtypeStruct((1, H), jnp.float32),
            jax.ShapeDtypeStruct((1, H), jnp.float32),
        ],
        scratch_shapes=[pltpu.VMEM((8, H), jnp.float32)],
    )(g, edge_attr, w1e, g1_2, beta1_2)

    out_e = pl.pallas_call(
        _edge_body,
        grid=(NEB,),
        in_specs=[
            pl.BlockSpec((TE, H), lambda i: (i, 0)),
            pl.BlockSpec((TE, DE), lambda i: (i, 0)),
            pl.BlockSpec((DE, H), lambda i: (0, 0)),
            pl.BlockSpec((1, H), lambda i: (0, 0)),
            pl.BlockSpec((1, H), lambda i: (0, 0)),
            pl.BlockSpec((H, H), lambda i: (0, 0)),
            pl.BlockSpec((1, H), lambda i: (0, 0)),
        ],
        out_specs=pl.BlockSpec((TE, H), lambda i: (i, 0)),
        out_shape=jax.ShapeDtypeStruct((E, H), jnp.float32),
    )(g, edge_attr, w1e, a, c, W1b, b1b.reshape(1, H))

    zeros = jnp.zeros((ZR, H), jnp.float32)
    agg = _sc_scatter(zeros, col, out_e)

    return pl.pallas_call(
        _node_body,
        out_shape=jax.ShapeDtypeStruct((N, D), jnp.float32),
    )(x, agg, W2a[:D], W2a[D:], b2a.reshape(1, H), g2.reshape(1, H),
      beta2.reshape(1, H), W2b, b2b.reshape(1, D))


# speedup vs baseline: 2.7486x; 1.2796x over previous
import functools

import jax
import jax.numpy as jnp
from jax import lax
from jax.experimental import pallas as pl
from jax.experimental.pallas import tpu as pltpu
from jax.experimental.pallas import tpu_sc as plsc

N = 10000
E = 320000
D = 128
DE = 16
H = 128

NC = 2
NS = 16
NW = NC * NS

LANE = 128
SUB = 2
BLK = SUB * LANE
NB = E // BLK
ZR = 1000

TE = 2000
NEB = E // TE

_sc_mesh = plsc.VectorSubcoreMesh(core_axis_name="c", subcore_axis_name="s")


@functools.partial(
    pl.kernel,
    mesh=_sc_mesh,
    out_type=jax.ShapeDtypeStruct((E, H), jnp.float32),
    scratch_types=[
        pltpu.VMEM((BLK,), jnp.int32),
        pltpu.VMEM((BLK, H), jnp.float32),
        pltpu.VMEM_SHARED((N, H), jnp.float32),
        pltpu.SemaphoreType.DMA,
    ],
)
def _sc_gather(p_hbm, row_hbm, out_hbm, idx_v, rows_v, p_sh, sem):
    cid = lax.axis_index("c")
    sid = lax.axis_index("s")
    wid = sid * NC + cid

    r0 = sid * ZR

    @pl.when(sid < N // ZR)
    def _():
        pltpu.sync_copy(p_hbm.at[pl.ds(r0, ZR)], p_sh.at[pl.ds(r0, ZR)])

    plsc.subcore_barrier()

    nb = (NB - wid + NW - 1) // NW

    def body(i, _):
        base = (wid + i * NW) * BLK
        pltpu.sync_copy(row_hbm.at[pl.ds(base, BLK)], idx_v)
        cps = [
            pltpu.async_copy(
                p_sh.at[idx_v.at[pl.ds(j * LANE, LANE)]],
                rows_v.at[pl.ds(j * LANE, LANE)],
                sem,
            )
            for j in range(SUB)
        ]
        for cp in cps:
            cp.wait()
        pltpu.sync_copy(rows_v, out_hbm.at[pl.ds(base, BLK)])
        return 0

    lax.fori_loop(0, nb, body, 0)


@functools.partial(
    pl.kernel,
    mesh=_sc_mesh,
    out_type=jax.ShapeDtypeStruct((NC, N, H), jnp.float32),
    scratch_types=[
        pltpu.VMEM((LANE,), jnp.int32),
        pltpu.VMEM((LANE,), jnp.int32),
        pltpu.VMEM((BLK, H), jnp.float32),
        pltpu.VMEM_SHARED((N, H), jnp.float32),
        pltpu.SemaphoreType.DMA,
    ],
)
def _sc_scatter(zeros_hbm, col_hbm, dat_hbm, out_hbm, idx_a, idx_b, dat_v,
                acc_sh, sem):
    cid = lax.axis_index("c")
    sid = lax.axis_index("s")

    r0 = sid * ZR

    @pl.when(sid < N // ZR)
    def _():
        pltpu.sync_copy(zeros_hbm, acc_sh.at[pl.ds(r0, ZR)])

    plsc.subcore_barrier()

    half = NB // NC
    nb = (half - sid + NS - 1) // NS

    def body(i, _):
        base = (cid * half + sid + i * NS) * BLK
        pltpu.sync_copy(col_hbm.at[pl.ds(base, LANE)], idx_a)
        pltpu.sync_copy(col_hbm.at[pl.ds(base + LANE, LANE)], idx_b)
        pltpu.sync_copy(dat_hbm.at[pl.ds(base, BLK)], dat_v)
        cps = [
            pltpu.async_copy(dat_v.at[pl.ds(0, LANE)],
                             acc_sh.at[idx_a], sem, add=True),
            pltpu.async_copy(dat_v.at[pl.ds(LANE, LANE)],
                             acc_sh.at[idx_b], sem, add=True),
        ]
        for cp in cps:
            cp.wait()
        return 0

    lax.fori_loop(0, nb, body, 0)
    plsc.subcore_barrier()

    @pl.when(sid < N // ZR)
    def _():
        pltpu.sync_copy(acc_sh.at[pl.ds(r0, ZR)],
                        out_hbm.at[cid].at[pl.ds(r0, ZR)])


def _prep_body(x_ref, w_ref, b_ref, p_ref):
    p_ref[...] = (
        jnp.dot(x_ref[...], w_ref[...], preferred_element_type=jnp.float32)
        + b_ref[...]
    )


def _stats_body(g_ref, ea_ref, w_ref, g1_ref, beta1_ref, a_ref, c_ref, acc):
    i = pl.program_id(0)
    h = g_ref[...] + jnp.dot(ea_ref[...], w_ref[...],
                             preferred_element_type=jnp.float32)
    s1 = jnp.sum(h, axis=0, keepdims=True)
    s2 = jnp.sum(h * h, axis=0, keepdims=True)

    @pl.when(i == 0)
    def _():
        acc[0:1, :] = s1
        acc[1:2, :] = s2

    @pl.when(i > 0)
    def _():
        acc[0:1, :] += s1
        acc[1:2, :] += s2

    @pl.when(i == NEB - 1)
    def _():
        mu = acc[0:1, :] * (1.0 / E)
        var = acc[1:2, :] * (1.0 / E) - mu * mu
        a = g1_ref[...] * lax.rsqrt(var + 1e-5)
        a_ref[...] = a
        c_ref[...] = beta1_ref[...] - mu * a


def _edge_body(g_ref, ea_ref, w_ref, a_ref, c_ref, w1b_ref, b1b_ref, o_ref):
    h = g_ref[...] + jnp.dot(ea_ref[...], w_ref[...],
                             preferred_element_type=jnp.float32)
    h = jnp.maximum(h * a_ref[...] + c_ref[...], 0.0)
    o_ref[...] = (
        jnp.dot(h, w1b_ref[...], preferred_element_type=jnp.float32)
        + b1b_ref[...]
    )


def _node_body(x_ref, agg_ref, w2x_ref, w2a_ref, b2a_ref, g2_ref,
               beta2_ref, w2b_ref, b2b_ref, o_ref):
    agg = agg_ref[0] + agg_ref[1]
    t = (
        jnp.dot(x_ref[...], w2x_ref[...], preferred_element_type=jnp.float32)
        + jnp.dot(agg, w2a_ref[...], preferred_element_type=jnp.float32)
        + b2a_ref[...]
    )
    mu = jnp.mean(t, axis=0, keepdims=True)
    var = jnp.mean(t * t, axis=0, keepdims=True) - mu * mu
    h = jnp.maximum((t - mu) * lax.rsqrt(var + 1e-5) * g2_ref[...]
                    + beta2_ref[...], 0.0)
    o_ref[...] = (
        jnp.dot(h, w2b_ref[...], preferred_element_type=jnp.float32)
        + b2b_ref[...]
    )


def kernel(x, edge_index, edge_attr, u, batch,
           W1a, b1a, g1, beta1, W1b, b1b,
           W2a, b2a, g2, beta2, W2b, b2b):
    row = edge_index[0]
    col = edge_index[1]

    w1x = W1a[:D]
    w1e = W1a[D:]
    b1a2 = b1a.reshape(1, H)
    g1_2 = g1.reshape(1, H)
    beta1_2 = beta1.reshape(1, H)

    p = pl.pallas_call(
        _prep_body,
        out_shape=jax.ShapeDtypeStruct((N, H), jnp.float32),
    )(x, w1x, b1a2)

    g = _sc_gather(p, row)

    a, c = pl.pallas_call(
        _stats_body,
        grid=(NEB,),
        in_specs=[
            pl.BlockSpec((TE, H), lambda i: (i, 0)),
            pl.BlockSpec((TE, DE), lambda i: (i, 0)),
            pl.BlockSpec((DE, H), lambda i: (0, 0)),
            pl.BlockSpec((1, H), lambda i: (0, 0)),
            pl.BlockSpec((1, H), lambda i: (0, 0)),
        ],
        out_specs=[
            pl.BlockSpec((1, H), lambda i: (0, 0)),
            pl.BlockSpec((1, H), lambda i: (0, 0)),
        ],
        out_shape=[
            jax.ShapeDtypeStruct((1, H), jnp.float32),
            jax.ShapeDtypeStruct((1, H), jnp.float32),
        ],
        scratch_shapes=[pltpu.VMEM((8, H), jnp.float32)],
    )(g, edge_attr, w1e, g1_2, beta1_2)

    out_e = pl.pallas_call(
        _edge_body,
        grid=(NEB,),
        in_specs=[
            pl.BlockSpec((TE, H), lambda i: (i, 0)),
            pl.BlockSpec((TE, DE), lambda i: (i, 0)),
            pl.BlockSpec((DE, H), lambda i: (0, 0)),
            pl.BlockSpec((1, H), lambda i: (0, 0)),
            pl.BlockSpec((1, H), lambda i: (0, 0)),
            pl.BlockSpec((H, H), lambda i: (0, 0)),
            pl.BlockSpec((1, H), lambda i: (0, 0)),
        ],
        out_specs=pl.BlockSpec((TE, H), lambda i: (i, 0)),
        out_shape=jax.ShapeDtypeStruct((E, H), jnp.float32),
    )(g, edge_attr, w1e, a, c, W1b, b1b.reshape(1, H))

    zeros = jnp.zeros((ZR, H), jnp.float32)
    agg = _sc_scatter(zeros, col, out_e)

    return pl.pallas_call(
        _node_body,
        out_shape=jax.ShapeDtypeStruct((N, D), jnp.float32),
    )(x, agg, W2a[:D], W2a[D:], b2a.reshape(1, H), g2.reshape(1, H),
      beta2.reshape(1, H), W2b, b2b.reshape(1, D))


# speedup vs baseline: 2.9692x; 1.0803x over previous
import functools

import jax
import jax.numpy as jnp
from jax import lax
from jax.experimental import pallas as pl
from jax.experimental.pallas import tpu as pltpu
from jax.experimental.pallas import tpu_sc as plsc

N = 10000
E = 320000
D = 128
DE = 16
H = 128

NC = 2
NS = 16
NW = NC * NS

KCH = 2
EC = E // KCH

LANE = 128
SUB = 2
BLK = SUB * LANE
NB = EC // BLK
ZR = 1000

TE = 2000
NEB = EC // TE

_sc_mesh = plsc.VectorSubcoreMesh(core_axis_name="c", subcore_axis_name="s")


@functools.partial(
    pl.kernel,
    mesh=_sc_mesh,
    out_type=jax.ShapeDtypeStruct((EC, H), jnp.float32),
    scratch_types=[
        pltpu.VMEM((BLK,), jnp.int32),
        pltpu.VMEM((BLK, H), jnp.float32),
        pltpu.VMEM_SHARED((N, H), jnp.float32),
        pltpu.SemaphoreType.DMA,
    ],
)
def _sc_gather(p_hbm, row_hbm, out_hbm, idx_v, rows_v, p_sh, sem):
    cid = lax.axis_index("c")
    sid = lax.axis_index("s")
    wid = sid * NC + cid

    r0 = sid * ZR

    @pl.when(sid < N // ZR)
    def _():
        pltpu.sync_copy(p_hbm.at[pl.ds(r0, ZR)], p_sh.at[pl.ds(r0, ZR)])

    plsc.subcore_barrier()

    nb = (NB - wid + NW - 1) // NW

    def body(i, _):
        base = (wid + i * NW) * BLK
        pltpu.sync_copy(row_hbm.at[pl.ds(base, BLK)], idx_v)
        cps = [
            pltpu.async_copy(
                p_sh.at[idx_v.at[pl.ds(j * LANE, LANE)]],
                rows_v.at[pl.ds(j * LANE, LANE)],
                sem,
            )
            for j in range(SUB)
        ]
        for cp in cps:
            cp.wait()
        pltpu.sync_copy(rows_v, out_hbm.at[pl.ds(base, BLK)])
        return 0

    lax.fori_loop(0, nb, body, 0)


@functools.partial(
    pl.kernel,
    mesh=_sc_mesh,
    out_type=jax.ShapeDtypeStruct((NC, N, H), jnp.float32),
    scratch_types=[
        pltpu.VMEM((LANE,), jnp.int32),
        pltpu.VMEM((LANE,), jnp.int32),
        pltpu.VMEM((BLK, H), jnp.float32),
        pltpu.VMEM_SHARED((N, H), jnp.float32),
        pltpu.SemaphoreType.DMA,
    ],
)
def _sc_scatter(zeros_hbm, col_hbm, dat_hbm, out_hbm, idx_a, idx_b, dat_v,
                acc_sh, sem):
    cid = lax.axis_index("c")
    sid = lax.axis_index("s")

    r0 = sid * ZR

    @pl.when(sid < N // ZR)
    def _():
        pltpu.sync_copy(zeros_hbm, acc_sh.at[pl.ds(r0, ZR)])

    plsc.subcore_barrier()

    half0 = (NB + 1) // NC
    start = cid * half0
    cnt = jnp.where(cid == 0, half0, NB - half0)
    nb = (cnt - sid + NS - 1) // NS

    def body(i, _):
        base = (start + sid + i * NS) * BLK
        pltpu.sync_copy(col_hbm.at[pl.ds(base, LANE)], idx_a)
        pltpu.sync_copy(col_hbm.at[pl.ds(base + LANE, LANE)], idx_b)
        pltpu.sync_copy(dat_hbm.at[pl.ds(base, BLK)], dat_v)
        cps = [
            pltpu.async_copy(dat_v.at[pl.ds(0, LANE)],
                             acc_sh.at[idx_a], sem, add=True),
            pltpu.async_copy(dat_v.at[pl.ds(LANE, LANE)],
                             acc_sh.at[idx_b], sem, add=True),
        ]
        for cp in cps:
            cp.wait()
        return 0

    lax.fori_loop(0, nb, body, 0)
    plsc.subcore_barrier()

    @pl.when(sid < N // ZR)
    def _():
        pltpu.sync_copy(acc_sh.at[pl.ds(r0, ZR)],
                        out_hbm.at[cid].at[pl.ds(r0, ZR)])


def _prep_body(x_ref, w_ref, b_ref, p_ref):
    p_ref[...] = (
        jnp.dot(x_ref[...], w_ref[...], preferred_element_type=jnp.float32)
        + b_ref[...]
    )


def _stats_body(g_ref, ea_ref, w_ref, s1_ref, s2_ref, acc):
    i = pl.program_id(0)
    h = g_ref[...] + jnp.dot(ea_ref[...], w_ref[...],
                             preferred_element_type=jnp.float32)
    s1 = jnp.sum(h, axis=0, keepdims=True)
    s2 = jnp.sum(h * h, axis=0, keepdims=True)

    @pl.when(i == 0)
    def _():
        acc[0:1, :] = s1
        acc[1:2, :] = s2

    @pl.when(i > 0)
    def _():
        acc[0:1, :] += s1
        acc[1:2, :] += s2

    @pl.when(i == NEB - 1)
    def _():
        s1_ref[...] = acc[0:1, :]
        s2_ref[...] = acc[1:2, :]


def _comb_body(s1_ref, s2_ref, g1_ref, beta1_ref, a_ref, c_ref):
    s1 = jnp.sum(s1_ref[...], axis=0, keepdims=True)
    s2 = jnp.sum(s2_ref[...], axis=0, keepdims=True)
    mu = s1 * (1.0 / E)
    var = s2 * (1.0 / E) - mu * mu
    a = g1_ref[...] * lax.rsqrt(var + 1e-5)
    a_ref[...] = a
    c_ref[...] = beta1_ref[...] - mu * a


def _edge_body(g_ref, ea_ref, w_ref, a_ref, c_ref, w1b_ref, b1b_ref, o_ref):
    h = g_ref[...] + jnp.dot(ea_ref[...], w_ref[...],
                             preferred_element_type=jnp.float32)
    h = jnp.maximum(h * a_ref[...] + c_ref[...], 0.0)
    o_ref[...] = (
        jnp.dot(h, w1b_ref[...], preferred_element_type=jnp.float32)
        + b1b_ref[...]
    )


def _node_body(x_ref, agg0_ref, agg1_ref, w2x_ref, w2a_ref, b2a_ref, g2_ref,
               beta2_ref, w2b_ref, b2b_ref, o_ref):
    agg = (agg0_ref[0] + agg0_ref[1]) + (agg1_ref[0] + agg1_ref[1])
    t = (
        jnp.dot(x_ref[...], w2x_ref[...], preferred_element_type=jnp.float32)
        + jnp.dot(agg, w2a_ref[...], preferred_element_type=jnp.float32)
        + b2a_ref[...]
    )
    mu = jnp.mean(t, axis=0, keepdims=True)
    var = jnp.mean(t * t, axis=0, keepdims=True) - mu * mu
    h = jnp.maximum((t - mu) * lax.rsqrt(var + 1e-5) * g2_ref[...]
                    + beta2_ref[...], 0.0)
    o_ref[...] = (
        jnp.dot(h, w2b_ref[...], preferred_element_type=jnp.float32)
        + b2b_ref[...]
    )


def kernel(x, edge_index, edge_attr, u, batch,
           W1a, b1a, g1, beta1, W1b, b1b,
           W2a, b2a, g2, beta2, W2b, b2b):
    row = edge_index[0]
    col = edge_index[1]

    w1x = W1a[:D]
    w1e = W1a[D:]
    b1a2 = b1a.reshape(1, H)
    g1_2 = g1.reshape(1, H)
    beta1_2 = beta1.reshape(1, H)

    p = pl.pallas_call(
        _prep_body,
        out_shape=jax.ShapeDtypeStruct((N, H), jnp.float32),
    )(x, w1x, b1a2)

    gs = [_sc_gather(p, row[k * EC:(k + 1) * EC]) for k in range(KCH)]

    stats_call = pl.pallas_call(
        _stats_body,
        grid=(NEB,),
        in_specs=[
            pl.BlockSpec((TE, H), lambda i: (i, 0)),
            pl.BlockSpec((TE, DE), lambda i: (i, 0)),
            pl.BlockSpec((DE, H), lambda i: (0, 0)),
        ],
        out_specs=[
            pl.BlockSpec((1, H), lambda i: (0, 0)),
            pl.BlockSpec((1, H), lambda i: (0, 0)),
        ],
        out_shape=[
            jax.ShapeDtypeStruct((1, H), jnp.float32),
            jax.ShapeDtypeStruct((1, H), jnp.float32),
        ],
        scratch_shapes=[pltpu.VMEM((8, H), jnp.float32)],
    )
    parts = [stats_call(gs[k], edge_attr[k * EC:(k + 1) * EC], w1e)
             for k in range(KCH)]
    s1 = jnp.concatenate([pr[0] for pr in parts], axis=0)
    s2 = jnp.concatenate([pr[1] for pr in parts], axis=0)

    a, c = pl.pallas_call(
        _comb_body,
        out_shape=[
            jax.ShapeDtypeStruct((1, H), jnp.float32),
            jax.ShapeDtypeStruct((1, H), jnp.float32),
        ],
    )(s1, s2, g1_2, beta1_2)

    edge_call = pl.pallas_call(
        _edge_body,
        grid=(NEB,),
        in_specs=[
            pl.BlockSpec((TE, H), lambda i: (i, 0)),
            pl.BlockSpec((TE, DE), lambda i: (i, 0)),
            pl.BlockSpec((DE, H), lambda i: (0, 0)),
            pl.BlockSpec((1, H), lambda i: (0, 0)),
            pl.BlockSpec((1, H), lambda i: (0, 0)),
            pl.BlockSpec((H, H), lambda i: (0, 0)),
            pl.BlockSpec((1, H), lambda i: (0, 0)),
        ],
        out_specs=pl.BlockSpec((TE, H), lambda i: (i, 0)),
        out_shape=jax.ShapeDtypeStruct((EC, H), jnp.float32),
    )
    outs = [edge_call(gs[k], edge_attr[k * EC:(k + 1) * EC], w1e, a, c,
                      W1b, b1b.reshape(1, H)) for k in range(KCH)]

    zeros = jnp.zeros((ZR, H), jnp.float32)
    aggs = [_sc_scatter(zeros, col[k * EC:(k + 1) * EC], outs[k])
            for k in range(KCH)]

    return pl.pallas_call(
        _node_body,
        out_shape=jax.ShapeDtypeStruct((N, D), jnp.float32),
    )(x, aggs[0], aggs[1], W2a[:D], W2a[D:], b2a.reshape(1, H),
      g2.reshape(1, H), beta2.reshape(1, H), W2b, b2b.reshape(1, D))


# speedup vs baseline: 3.2548x; 1.0962x over previous
import functools

import jax
import jax.numpy as jnp
from jax import lax
from jax.experimental import pallas as pl
from jax.experimental.pallas import tpu as pltpu
from jax.experimental.pallas import tpu_sc as plsc

N = 10000
E = 320000
D = 128
DE = 16
H = 128

NC = 2
NS = 16
NW = NC * NS

KCH = 2
EC = E // KCH

LANE = 128
SUB = 2
BLK = SUB * LANE
NB = EC // BLK
ZR = 1000

TE = 4000
NEB = EC // TE

_sc_mesh = plsc.VectorSubcoreMesh(core_axis_name="c", subcore_axis_name="s")


@functools.partial(
    pl.kernel,
    mesh=_sc_mesh,
    out_type=jax.ShapeDtypeStruct((EC, H), jnp.float32),
    scratch_types=[
        pltpu.VMEM((BLK,), jnp.int32),
        pltpu.VMEM((BLK, H), jnp.float32),
        pltpu.VMEM_SHARED((N, H), jnp.float32),
        pltpu.SemaphoreType.DMA,
    ],
)
def _sc_gather(p_hbm, row_hbm, out_hbm, idx_v, rows_v, p_sh, sem):
    cid = lax.axis_index("c")
    sid = lax.axis_index("s")
    wid = sid * NC + cid

    r0 = sid * ZR

    @pl.when(sid < N // ZR)
    def _():
        pltpu.sync_copy(p_hbm.at[pl.ds(r0, ZR)], p_sh.at[pl.ds(r0, ZR)])

    plsc.subcore_barrier()

    nb = (NB - wid + NW - 1) // NW

    def body(i, _):
        base = (wid + i * NW) * BLK
        pltpu.sync_copy(row_hbm.at[pl.ds(base, BLK)], idx_v)
        cps = [
            pltpu.async_copy(
                p_sh.at[idx_v.at[pl.ds(j * LANE, LANE)]],
                rows_v.at[pl.ds(j * LANE, LANE)],
                sem,
            )
            for j in range(SUB)
        ]
        for cp in cps:
            cp.wait()
        pltpu.sync_copy(rows_v, out_hbm.at[pl.ds(base, BLK)])
        return 0

    lax.fori_loop(0, nb, body, 0)


@functools.partial(
    pl.kernel,
    mesh=_sc_mesh,
    out_type=jax.ShapeDtypeStruct((NC, N, H), jnp.float32),
    scratch_types=[
        pltpu.VMEM((LANE,), jnp.int32),
        pltpu.VMEM((LANE,), jnp.int32),
        pltpu.VMEM((BLK, H), jnp.float32),
        pltpu.VMEM_SHARED((N, H), jnp.float32),
        pltpu.SemaphoreType.DMA,
    ],
)
def _sc_scatter(zeros_hbm, col_hbm, dat_hbm, out_hbm, idx_a, idx_b, dat_v,
                acc_sh, sem):
    cid = lax.axis_index("c")
    sid = lax.axis_index("s")

    r0 = sid * ZR

    @pl.when(sid < N // ZR)
    def _():
        pltpu.sync_copy(zeros_hbm, acc_sh.at[pl.ds(r0, ZR)])

    plsc.subcore_barrier()

    half0 = (NB + 1) // NC
    start = cid * half0
    cnt = jnp.where(cid == 0, half0, NB - half0)
    nb = (cnt - sid + NS - 1) // NS

    def body(i, _):
        base = (start + sid + i * NS) * BLK
        pltpu.sync_copy(col_hbm.at[pl.ds(base, LANE)], idx_a)
        pltpu.sync_copy(col_hbm.at[pl.ds(base + LANE, LANE)], idx_b)
        pltpu.sync_copy(dat_hbm.at[pl.ds(base, BLK)], dat_v)
        cps = [
            pltpu.async_copy(dat_v.at[pl.ds(0, LANE)],
                             acc_sh.at[idx_a], sem, add=True),
            pltpu.async_copy(dat_v.at[pl.ds(LANE, LANE)],
                             acc_sh.at[idx_b], sem, add=True),
        ]
        for cp in cps:
            cp.wait()
        return 0

    lax.fori_loop(0, nb, body, 0)
    plsc.subcore_barrier()

    @pl.when(sid < N // ZR)
    def _():
        pltpu.sync_copy(acc_sh.at[pl.ds(r0, ZR)],
                        out_hbm.at[cid].at[pl.ds(r0, ZR)])


def _prep_body(x_ref, w_ref, b_ref, p_ref):
    p_ref[...] = (
        jnp.dot(x_ref[...], w_ref[...], preferred_element_type=jnp.float32)
        + b_ref[...]
    )


def _stats_body(g_ref, ea_ref, w_ref, s1_ref, s2_ref, acc):
    i = pl.program_id(0)
    h = g_ref[...] + jnp.dot(ea_ref[...], w_ref[...],
                             preferred_element_type=jnp.float32)
    s1 = jnp.sum(h, axis=0, keepdims=True)
    s2 = jnp.sum(h * h, axis=0, keepdims=True)

    @pl.when(i == 0)
    def _():
        acc[0:1, :] = s1
        acc[1:2, :] = s2

    @pl.when(i > 0)
    def _():
        acc[0:1, :] += s1
        acc[1:2, :] += s2

    @pl.when(i == NEB - 1)
    def _():
        s1_ref[...] = acc[0:1, :]
        s2_ref[...] = acc[1:2, :]


def _comb_body(s1_ref, s2_ref, g1_ref, beta1_ref, a_ref, c_ref):
    s1 = jnp.sum(s1_ref[...], axis=0, keepdims=True)
    s2 = jnp.sum(s2_ref[...], axis=0, keepdims=True)
    mu = s1 * (1.0 / E)
    var = s2 * (1.0 / E) - mu * mu
    a = g1_ref[...] * lax.rsqrt(var + 1e-5)
    a_ref[...] = a
    c_ref[...] = beta1_ref[...] - mu * a


def _edge_body(g_ref, ea_ref, w_ref, a_ref, c_ref, w1b_ref, b1b_ref, o_ref):
    h = g_ref[...] + jnp.dot(ea_ref[...], w_ref[...],
                             preferred_element_type=jnp.float32)
    h = jnp.maximum(h * a_ref[...] + c_ref[...], 0.0)
    o_ref[...] = (
        jnp.dot(h, w1b_ref[...], preferred_element_type=jnp.float32)
        + b1b_ref[...]
    )


def _node_body(x_ref, agg0_ref, agg1_ref, w2x_ref, w2a_ref, b2a_ref, g2_ref,
               beta2_ref, w2b_ref, b2b_ref, o_ref):
    agg = (agg0_ref[0] + agg0_ref[1]) + (agg1_ref[0] + agg1_ref[1])
    t = (
        jnp.dot(x_ref[...], w2x_ref[...], preferred_element_type=jnp.float32)
        + jnp.dot(agg, w2a_ref[...], preferred_element_type=jnp.float32)
        + b2a_ref[...]
    )
    mu = jnp.mean(t, axis=0, keepdims=True)
    var = jnp.mean(t * t, axis=0, keepdims=True) - mu * mu
    h = jnp.maximum((t - mu) * lax.rsqrt(var + 1e-5) * g2_ref[...]
                    + beta2_ref[...], 0.0)
    o_ref[...] = (
        jnp.dot(h, w2b_ref[...], preferred_element_type=jnp.float32)
        + b2b_ref[...]
    )


def kernel(x, edge_index, edge_attr, u, batch,
           W1a, b1a, g1, beta1, W1b, b1b,
           W2a, b2a, g2, beta2, W2b, b2b):
    row = edge_index[0]
    col = edge_index[1]

    w1x = W1a[:D]
    w1e = W1a[D:]
    b1a2 = b1a.reshape(1, H)
    g1_2 = g1.reshape(1, H)
    beta1_2 = beta1.reshape(1, H)

    p = pl.pallas_call(
        _prep_body,
        out_shape=jax.ShapeDtypeStruct((N, H), jnp.float32),
    )(x, w1x, b1a2)

    gs = [_sc_gather(p, row[k * EC:(k + 1) * EC]) for k in range(KCH)]

    stats_call = pl.pallas_call(
        _stats_body,
        grid=(NEB,),
        in_specs=[
            pl.BlockSpec((TE, H), lambda i: (i, 0)),
            pl.BlockSpec((TE, DE), lambda i: (i, 0)),
            pl.BlockSpec((DE, H), lambda i: (0, 0)),
        ],
        out_specs=[
            pl.BlockSpec((1, H), lambda i: (0, 0)),
            pl.BlockSpec((1, H), lambda i: (0, 0)),
        ],
        out_shape=[
            jax.ShapeDtypeStruct((1, H), jnp.float32),
            jax.ShapeDtypeStruct((1, H), jnp.float32),
        ],
        scratch_shapes=[pltpu.VMEM((8, H), jnp.float32)],
    )
    parts = [stats_call(gs[k], edge_attr[k * EC:(k + 1) * EC], w1e)
             for k in range(KCH)]
    s1 = jnp.concatenate([pr[0] for pr in parts], axis=0)
    s2 = jnp.concatenate([pr[1] for pr in parts], axis=0)

    a, c = pl.pallas_call(
        _comb_body,
        out_shape=[
            jax.ShapeDtypeStruct((1, H), jnp.float32),
            jax.ShapeDtypeStruct((1, H), jnp.float32),
        ],
    )(s1, s2, g1_2, beta1_2)

    edge_call = pl.pallas_call(
        _edge_body,
        grid=(NEB,),
        in_specs=[
            pl.BlockSpec((TE, H), lambda i: (i, 0)),
            pl.BlockSpec((TE, DE), lambda i: (i, 0)),
            pl.BlockSpec((DE, H), lambda i: (0, 0)),
            pl.BlockSpec((1, H), lambda i: (0, 0)),
            pl.BlockSpec((1, H), lambda i: (0, 0)),
            pl.BlockSpec((H, H), lambda i: (0, 0)),
            pl.BlockSpec((1, H), lambda i: (0, 0)),
        ],
        out_specs=pl.BlockSpec((TE, H), lambda i: (i, 0)),
        out_shape=jax.ShapeDtypeStruct((EC, H), jnp.float32),
    )
    outs = [edge_call(gs[k], edge_attr[k * EC:(k + 1) * EC], w1e, a, c,
                      W1b, b1b.reshape(1, H)) for k in range(KCH)]

    zeros = jnp.zeros((ZR, H), jnp.float32)
    aggs = [_sc_scatter(zeros, col[k * EC:(k + 1) * EC], outs[k])
            for k in range(KCH)]

    return pl.pallas_call(
        _node_body,
        out_shape=jax.ShapeDtypeStruct((N, D), jnp.float32),
    )(x, aggs[0], aggs[1], W2a[:D], W2a[D:], b2a.reshape(1, H),
      g2.reshape(1, H), beta2.reshape(1, H), W2b, b2b.reshape(1, D))


# speedup vs baseline: 3.3559x; 1.0310x over previous
import functools

import jax
import jax.numpy as jnp
from jax import lax
from jax.experimental import pallas as pl
from jax.experimental.pallas import tpu as pltpu
from jax.experimental.pallas import tpu_sc as plsc

N = 10000
E = 320000
D = 128
DE = 16
H = 128

NC = 2
NS = 16
NW = NC * NS

KCH = 2
EC = E // KCH

LANE = 128
SUB = 2
BLK = SUB * LANE
NB = EC // BLK
ZR = 1000

TE = 8000
NEB = EC // TE

_sc_mesh = plsc.VectorSubcoreMesh(core_axis_name="c", subcore_axis_name="s")


@functools.partial(
    pl.kernel,
    mesh=_sc_mesh,
    out_type=jax.ShapeDtypeStruct((EC, H), jnp.float32),
    scratch_types=[
        pltpu.VMEM((BLK,), jnp.int32),
        pltpu.VMEM((BLK, H), jnp.float32),
        pltpu.VMEM_SHARED((N, H), jnp.float32),
        pltpu.SemaphoreType.DMA,
    ],
)
def _sc_gather(p_hbm, row_hbm, out_hbm, idx_v, rows_v, p_sh, sem):
    cid = lax.axis_index("c")
    sid = lax.axis_index("s")
    wid = sid * NC + cid

    r0 = sid * ZR

    @pl.when(sid < N // ZR)
    def _():
        pltpu.sync_copy(p_hbm.at[pl.ds(r0, ZR)], p_sh.at[pl.ds(r0, ZR)])

    plsc.subcore_barrier()

    nb = (NB - wid + NW - 1) // NW

    def body(i, _):
        base = (wid + i * NW) * BLK
        pltpu.sync_copy(row_hbm.at[pl.ds(base, BLK)], idx_v)
        cps = [
            pltpu.async_copy(
                p_sh.at[idx_v.at[pl.ds(j * LANE, LANE)]],
                rows_v.at[pl.ds(j * LANE, LANE)],
                sem,
            )
            for j in range(SUB)
        ]
        for cp in cps:
            cp.wait()
        pltpu.sync_copy(rows_v, out_hbm.at[pl.ds(base, BLK)])
        return 0

    lax.fori_loop(0, nb, body, 0)


@functools.partial(
    pl.kernel,
    mesh=_sc_mesh,
    out_type=jax.ShapeDtypeStruct((NC, N, H), jnp.float32),
    scratch_types=[
        pltpu.VMEM((LANE,), jnp.int32),
        pltpu.VMEM((LANE,), jnp.int32),
        pltpu.VMEM((BLK, H), jnp.float32),
        pltpu.VMEM_SHARED((N, H), jnp.float32),
        pltpu.SemaphoreType.DMA,
    ],
)
def _sc_scatter(zeros_hbm, col_hbm, dat_hbm, out_hbm, idx_a, idx_b, dat_v,
                acc_sh, sem):
    cid = lax.axis_index("c")
    sid = lax.axis_index("s")

    r0 = sid * ZR

    @pl.when(sid < N // ZR)
    def _():
        pltpu.sync_copy(zeros_hbm, acc_sh.at[pl.ds(r0, ZR)])

    plsc.subcore_barrier()

    half0 = (NB + 1) // NC
    start = cid * half0
    cnt = jnp.where(cid == 0, half0, NB - half0)
    nb = (cnt - sid + NS - 1) // NS

    def body(i, _):
        base = (start + sid + i * NS) * BLK
        pltpu.sync_copy(col_hbm.at[pl.ds(base, LANE)], idx_a)
        pltpu.sync_copy(col_hbm.at[pl.ds(base + LANE, LANE)], idx_b)
        pltpu.sync_copy(dat_hbm.at[pl.ds(base, BLK)], dat_v)
        cps = [
            pltpu.async_copy(dat_v.at[pl.ds(0, LANE)],
                             acc_sh.at[idx_a], sem, add=True),
            pltpu.async_copy(dat_v.at[pl.ds(LANE, LANE)],
                             acc_sh.at[idx_b], sem, add=True),
        ]
        for cp in cps:
            cp.wait()
        return 0

    lax.fori_loop(0, nb, body, 0)
    plsc.subcore_barrier()

    @pl.when(sid < N // ZR)
    def _():
        pltpu.sync_copy(acc_sh.at[pl.ds(r0, ZR)],
                        out_hbm.at[cid].at[pl.ds(r0, ZR)])


def _prep_body(x_ref, w_ref, b_ref, p_ref):
    p_ref[...] = (
        jnp.dot(x_ref[...], w_ref[...], preferred_element_type=jnp.float32)
        + b_ref[...]
    )


def _stats_body(g_ref, ea_ref, w_ref, s1_ref, s2_ref, acc):
    i = pl.program_id(0)
    h = g_ref[...] + jnp.dot(ea_ref[...], w_ref[...],
                             preferred_element_type=jnp.float32)
    s1 = jnp.sum(h, axis=0, keepdims=True)
    s2 = jnp.sum(h * h, axis=0, keepdims=True)

    @pl.when(i == 0)
    def _():
        acc[0:1, :] = s1
        acc[1:2, :] = s2

    @pl.when(i > 0)
    def _():
        acc[0:1, :] += s1
        acc[1:2, :] += s2

    @pl.when(i == NEB - 1)
    def _():
        s1_ref[...] = acc[0:1, :]
        s2_ref[...] = acc[1:2, :]


def _comb_body(s1_ref, s2_ref, g1_ref, beta1_ref, a_ref, c_ref):
    s1 = jnp.sum(s1_ref[...], axis=0, keepdims=True)
    s2 = jnp.sum(s2_ref[...], axis=0, keepdims=True)
    mu = s1 * (1.0 / E)
    var = s2 * (1.0 / E) - mu * mu
    a = g1_ref[...] * lax.rsqrt(var + 1e-5)
    a_ref[...] = a
    c_ref[...] = beta1_ref[...] - mu * a


def _edge_body(g_ref, ea_ref, w_ref, a_ref, c_ref, w1b_ref, b1b_ref, o_ref):
    h = g_ref[...] + jnp.dot(ea_ref[...], w_ref[...],
                             preferred_element_type=jnp.float32)
    h = jnp.maximum(h * a_ref[...] + c_ref[...], 0.0)
    o_ref[...] = (
        jnp.dot(h, w1b_ref[...], preferred_element_type=jnp.float32)
        + b1b_ref[...]
    )


def _node_body(x_ref, agg0_ref, agg1_ref, w2x_ref, w2a_ref, b2a_ref, g2_ref,
               beta2_ref, w2b_ref, b2b_ref, o_ref):
    agg = (agg0_ref[0] + agg0_ref[1]) + (agg1_ref[0] + agg1_ref[1])
    t = (
        jnp.dot(x_ref[...], w2x_ref[...], preferred_element_type=jnp.float32)
        + jnp.dot(agg, w2a_ref[...], preferred_element_type=jnp.float32)
        + b2a_ref[...]
    )
    mu = jnp.mean(t, axis=0, keepdims=True)
    var = jnp.mean(t * t, axis=0, keepdims=True) - mu * mu
    h = jnp.maximum((t - mu) * lax.rsqrt(var + 1e-5) * g2_ref[...]
                    + beta2_ref[...], 0.0)
    o_ref[...] = (
        jnp.dot(h, w2b_ref[...], preferred_element_type=jnp.float32)
        + b2b_ref[...]
    )


def kernel(x, edge_index, edge_attr, u, batch,
           W1a, b1a, g1, beta1, W1b, b1b,
           W2a, b2a, g2, beta2, W2b, b2b):
    row = edge_index[0]
    col = edge_index[1]

    w1x = W1a[:D]
    w1e = W1a[D:]
    b1a2 = b1a.reshape(1, H)
    g1_2 = g1.reshape(1, H)
    beta1_2 = beta1.reshape(1, H)

    p = pl.pallas_call(
        _prep_body,
        out_shape=jax.ShapeDtypeStruct((N, H), jnp.float32),
    )(x, w1x, b1a2)

    gs = [_sc_gather(p, row[k * EC:(k + 1) * EC]) for k in range(KCH)]

    stats_call = pl.pallas_call(
        _stats_body,
        grid=(NEB,),
        in_specs=[
            pl.BlockSpec((TE, H), lambda i: (i, 0)),
            pl.BlockSpec((TE, DE), lambda i: (i, 0)),
            pl.BlockSpec((DE, H), lambda i: (0, 0)),
        ],
        out_specs=[
            pl.BlockSpec((1, H), lambda i: (0, 0)),
            pl.BlockSpec((1, H), lambda i: (0, 0)),
        ],
        out_shape=[
            jax.ShapeDtypeStruct((1, H), jnp.float32),
            jax.ShapeDtypeStruct((1, H), jnp.float32),
        ],
        scratch_shapes=[pltpu.VMEM((8, H), jnp.float32)],
    )
    parts = [stats_call(gs[k], edge_attr[k * EC:(k + 1) * EC], w1e)
             for k in range(KCH)]
    s1 = jnp.concatenate([pr[0] for pr in parts], axis=0)
    s2 = jnp.concatenate([pr[1] for pr in parts], axis=0)

    a, c = pl.pallas_call(
        _comb_body,
        out_shape=[
            jax.ShapeDtypeStruct((1, H), jnp.float32),
            jax.ShapeDtypeStruct((1, H), jnp.float32),
        ],
    )(s1, s2, g1_2, beta1_2)

    edge_call = pl.pallas_call(
        _edge_body,
        grid=(NEB,),
        in_specs=[
            pl.BlockSpec((TE, H), lambda i: (i, 0)),
            pl.BlockSpec((TE, DE), lambda i: (i, 0)),
            pl.BlockSpec((DE, H), lambda i: (0, 0)),
            pl.BlockSpec((1, H), lambda i: (0, 0)),
            pl.BlockSpec((1, H), lambda i: (0, 0)),
            pl.BlockSpec((H, H), lambda i: (0, 0)),
            pl.BlockSpec((1, H), lambda i: (0, 0)),
        ],
        out_specs=pl.BlockSpec((TE, H), lambda i: (i, 0)),
        out_shape=jax.ShapeDtypeStruct((EC, H), jnp.float32),
    )
    outs = [edge_call(gs[k], edge_attr[k * EC:(k + 1) * EC], w1e, a, c,
                      W1b, b1b.reshape(1, H)) for k in range(KCH)]

    zeros = jnp.zeros((ZR, H), jnp.float32)
    aggs = [_sc_scatter(zeros, col[k * EC:(k + 1) * EC], outs[k])
            for k in range(KCH)]

    return pl.pallas_call(
        _node_body,
        out_shape=jax.ShapeDtypeStruct((N, D), jnp.float32),
    )(x, aggs[0], aggs[1], W2a[:D], W2a[D:], b2a.reshape(1, H),
      g2.reshape(1, H), beta2.reshape(1, H), W2b, b2b.reshape(1, D))
